# Initial kernel scaffold; baseline (speedup 1.0000x reference)
import jax
import jax.numpy as jnp
from jax.experimental import pallas as pl


def kernel(x, Wg, bg, Wb, bb, Wr1, br1, Wt1, Wr2, br2, Wt2, Wr3, br3, Wt3, Wr4, br4, Wt4, Wr5, br5, Wt5, Wr6, br6, Wt6, Wr7, br7, Wt7):
    raise NotImplementedError("write your pallas kernel here")


# speedup vs baseline: 9.5556x; 9.5556x over previous
import jax
import jax.numpy as jnp
from jax import lax
from jax.experimental import pallas as pl

_CH = 62
_CHP = 64
_TOPK = 10


def _mm(a, b):
    return jnp.dot(a, b, preferred_element_type=jnp.float32)


def _bmm(a, b):
    return lax.dot_general(a, b, (((2,), (1,)), ((0,), (0,))),
                           preferred_element_type=jnp.float32)


def _body(refs, bb):
    (x_ref, Wg, bg, Wb, bbias,
     Wr1, br1, Wt1, Wr2, br2, Wt2, Wr3, br3, Wt3, Wr4, br4, Wt4,
     Wr5, br5, Wt5, Wr6, br6, Wt6, Wr7, br7, Wt7, o_ref) = refs

    xv = x_ref[...]
    xp = jnp.concatenate(
        [xv, jnp.zeros((bb, _CHP - _CH, xv.shape[2]), jnp.float32)], axis=1)
    x2 = xp.reshape(bb * _CHP, xv.shape[2])

    xm = jnp.tanh(_mm(x2, Wg[...]) + bg[...])
    xa = jnp.tanh(_mm(x2, Wb[...]) + bbias[...])
    xa3 = xa.reshape(bb, _CHP, xa.shape[1])

    adj = lax.dot_general(xa3, xa3, (((2,), (2,)), ((0,), (0,))),
                          preferred_element_type=jnp.float32)
    colidx = lax.broadcasted_iota(jnp.int32, (bb, _CHP, _CHP), 2)
    adj = jnp.where(colidx < _CH, adj, -1e30)
    adj = adj - jnp.max(adj, axis=2, keepdims=True)
    e = jnp.exp(adj)
    adj = e / jnp.sum(e, axis=2, keepdims=True)

    work = adj
    keep = jnp.zeros_like(adj)
    for _ in range(_TOPK):
        m = jnp.max(work, axis=2, keepdims=True)
        ismax = work == m
        first = jnp.min(jnp.where(ismax, colidx, _CHP), axis=2, keepdims=True)
        sel = colidx == first
        keep = jnp.where(sel, 1.0, keep)
        work = jnp.where(sel, -1.0, work)
    adjm = adj * keep

    def gconv(h2, Wr, br, Wt):
        f = h2.shape[1]
        h3 = h2.reshape(bb, _CHP, f)
        a2 = _bmm(adjm, h3).reshape(bb * _CHP, f)
        return _mm(a2, Wr[...]) + br[...] + _mm(h2, Wt[...])

    h = jax.nn.relu(gconv(x2, Wr1, br1, Wt1))
    h = h + jax.nn.relu(gconv(h, Wr2, br2, Wt2))
    h = h + jax.nn.relu(gconv(h, Wr3, br3, Wt3))
    h = h + jax.nn.relu(gconv(h, Wr4, br4, Wt4))
    h = h + jax.nn.relu(gconv(h, Wr5, br5, Wt5))
    h = h + jax.nn.relu(gconv(h, Wr6, br6, Wt6))
    h = jax.nn.relu(gconv(h, Wr7, br7, Wt7))

    out = (h * xm).reshape(bb, _CHP, xm.shape[1])
    o_ref[...] = out[:, :_CH, :]


def kernel(x, Wg, bg, Wb, bb, Wr1, br1, Wt1, Wr2, br2, Wt2, Wr3, br3, Wt3,
           Wr4, br4, Wt4, Wr5, br5, Wt5, Wr6, br6, Wt6, Wr7, br7, Wt7):
    B = x.shape[0]
    BB = 64
    while B % BB:
        BB //= 2
    grid = (B // BB,)

    weights = [Wg, bg.reshape(1, -1), Wb, bb.reshape(1, -1)]
    for Wr, br, Wt in ((Wr1, br1, Wt1), (Wr2, br2, Wt2), (Wr3, br3, Wt3),
                       (Wr4, br4, Wt4), (Wr5, br5, Wt5), (Wr6, br6, Wt6),
                       (Wr7, br7, Wt7)):
        weights += [Wr, br.reshape(1, -1), Wt]

    in_specs = [pl.BlockSpec((BB, _CH, x.shape[2]), lambda i: (i, 0, 0))]
    for w in weights:
        in_specs.append(pl.BlockSpec(w.shape, lambda i: (0,) * w.ndim))

    out_f = Wr7.shape[1]
    fn = lambda *refs: _body(refs, BB)
    return pl.pallas_call(
        fn,
        grid=grid,
        in_specs=in_specs,
        out_specs=pl.BlockSpec((BB, _CH, out_f), lambda i: (i, 0, 0)),
        out_shape=jax.ShapeDtypeStruct((B, _CH, out_f), jnp.float32),
    )(x, *weights)


# speedup vs baseline: 15.2630x; 1.5973x over previous
import jax
import jax.numpy as jnp
from jax import lax
from jax.experimental import pallas as pl

_CH = 62
_CHP = 64
_TOPK = 10


def _mm(a, b):
    return jnp.dot(a, b, preferred_element_type=jnp.float32)


def _bmm(a, b):
    return lax.dot_general(a, b, (((2,), (1,)), ((0,), (0,))),
                           preferred_element_type=jnp.float32)


def _body(refs, bb):
    (x_ref, Wg, bg, Wb, bbias,
     Wr1, br1, Wt1, Wr2, br2, Wt2, Wr3, br3, Wt3, Wr4, br4, Wt4,
     Wr5, br5, Wt5, Wr6, br6, Wt6, Wr7, br7, Wt7, o_ref) = refs

    xv = x_ref[...]
    xp = jnp.concatenate(
        [xv, jnp.zeros((bb, _CHP - _CH, xv.shape[2]), jnp.float32)], axis=1)
    x2 = xp.reshape(bb * _CHP, xv.shape[2])

    xm = jnp.tanh(_mm(x2, Wg[...]) + bg[...])
    xa = jnp.tanh(_mm(x2, Wb[...]) + bbias[...])
    xa3 = xa.reshape(bb, _CHP, xa.shape[1])

    adj = lax.dot_general(xa3, xa3, (((2,), (2,)), ((0,), (0,))),
                          preferred_element_type=jnp.float32)
    colidx = lax.broadcasted_iota(jnp.int32, (1, 1, _CHP), 2)
    adj = jnp.where(colidx < _CH, adj, -1e30)
    adj = adj - jnp.max(adj, axis=2, keepdims=True)
    e = jnp.exp(adj)
    work = e / jnp.sum(e, axis=2, keepdims=True)

    for _ in range(_TOPK):
        m = jnp.max(work, axis=2, keepdims=True)
        m = jnp.where(m >= 0.0, m, 100.0)
        work = jnp.where(work == m, work - 2.0, work)
    adjm = jnp.where(work < 0.0, work + 2.0, 0.0)

    def gconv(h2, Wr, br, Wt):
        f = h2.shape[1]
        h3 = h2.reshape(bb, _CHP, f)
        a2 = _bmm(adjm, h3).reshape(bb * _CHP, f)
        return _mm(a2, Wr[...]) + br[...] + _mm(h2, Wt[...])

    h = jax.nn.relu(gconv(x2, Wr1, br1, Wt1))
    h = h + jax.nn.relu(gconv(h, Wr2, br2, Wt2))
    h = h + jax.nn.relu(gconv(h, Wr3, br3, Wt3))
    h = h + jax.nn.relu(gconv(h, Wr4, br4, Wt4))
    h = h + jax.nn.relu(gconv(h, Wr5, br5, Wt5))
    h = h + jax.nn.relu(gconv(h, Wr6, br6, Wt6))
    h = jax.nn.relu(gconv(h, Wr7, br7, Wt7))

    out = (h * xm).reshape(bb, _CHP, xm.shape[1])
    o_ref[...] = out[:, :_CH, :]


def kernel(x, Wg, bg, Wb, bb, Wr1, br1, Wt1, Wr2, br2, Wt2, Wr3, br3, Wt3,
           Wr4, br4, Wt4, Wr5, br5, Wt5, Wr6, br6, Wt6, Wr7, br7, Wt7):
    B = x.shape[0]
    BB = 64
    while B % BB:
        BB //= 2
    grid = (B // BB,)

    weights = [Wg, bg.reshape(1, -1), Wb, bb.reshape(1, -1)]
    for Wr, br, Wt in ((Wr1, br1, Wt1), (Wr2, br2, Wt2), (Wr3, br3, Wt3),
                       (Wr4, br4, Wt4), (Wr5, br5, Wt5), (Wr6, br6, Wt6),
                       (Wr7, br7, Wt7)):
        weights += [Wr, br.reshape(1, -1), Wt]

    in_specs = [pl.BlockSpec((BB, _CH, x.shape[2]), lambda i: (i, 0, 0))]
    for w in weights:
        in_specs.append(pl.BlockSpec(w.shape, lambda i: (0,) * w.ndim))

    out_f = Wr7.shape[1]
    fn = lambda *refs: _body(refs, BB)
    return pl.pallas_call(
        fn,
        grid=grid,
        in_specs=in_specs,
        out_specs=pl.BlockSpec((BB, _CH, out_f), lambda i: (i, 0, 0)),
        out_shape=jax.ShapeDtypeStruct((B, _CH, out_f), jnp.float32),
    )(x, *weights)


# speedup vs baseline: 15.6300x; 1.0240x over previous
import jax
import jax.numpy as jnp
from jax import lax
from jax.experimental import pallas as pl

_CH = 62
_CHP = 64
_TOPK = 10


def _mm(a, b):
    return jnp.dot(a, b, preferred_element_type=jnp.float32)


def _bmm(a, b):
    return lax.dot_general(a, b, (((2,), (1,)), ((0,), (0,))),
                           preferred_element_type=jnp.float32)


def _body(refs, bb):
    (x_ref, Wg, bg, Wb, bbias,
     Wr1, br1, Wt1, Wr2, br2, Wt2, Wr3, br3, Wt3, Wr4, br4, Wt4,
     Wr5, br5, Wt5, Wr6, br6, Wt6, Wr7, br7, Wt7, o_ref) = refs

    xv = x_ref[...]
    xp = jnp.concatenate(
        [xv, jnp.zeros((bb, _CHP - _CH, xv.shape[2]), jnp.float32)], axis=1)
    x2 = xp.reshape(bb * _CHP, xv.shape[2])

    xm = jnp.tanh(_mm(x2, Wg[...]) + bg[...])
    xa = jnp.tanh(_mm(x2, Wb[...]) + bbias[...])
    xa3 = xa.reshape(bb, _CHP, xa.shape[1])

    adj = lax.dot_general(xa3, xa3, (((2,), (2,)), ((0,), (0,))),
                          preferred_element_type=jnp.float32)
    colidx = lax.broadcasted_iota(jnp.int32, (1, 1, _CHP), 2)
    e = jnp.where(colidx < _CH, jnp.exp(adj), 0.0)
    work = e / jnp.sum(e, axis=2, keepdims=True)

    for _ in range(_TOPK):
        m = jnp.max(work, axis=2, keepdims=True)
        m = jnp.where(m >= 0.0, m, 100.0)
        work = jnp.where(work == m, work - 2.0, work)
    adjm = jnp.where(work < 0.0, work + 2.0, 0.0)

    def gconv(h2, Wr, br, Wt):
        f = h2.shape[1]
        h3 = h2.reshape(bb, _CHP, f)
        a2 = _bmm(adjm, h3).reshape(bb * _CHP, f)
        return _mm(a2, Wr[...]) + br[...] + _mm(h2, Wt[...])

    h = jax.nn.relu(gconv(x2, Wr1, br1, Wt1))
    h = h + jax.nn.relu(gconv(h, Wr2, br2, Wt2))
    h = h + jax.nn.relu(gconv(h, Wr3, br3, Wt3))
    h = h + jax.nn.relu(gconv(h, Wr4, br4, Wt4))
    h = h + jax.nn.relu(gconv(h, Wr5, br5, Wt5))
    h = h + jax.nn.relu(gconv(h, Wr6, br6, Wt6))
    h = jax.nn.relu(gconv(h, Wr7, br7, Wt7))

    out = (h * xm).reshape(bb, _CHP, xm.shape[1])
    o_ref[...] = out[:, :_CH, :]


def kernel(x, Wg, bg, Wb, bb, Wr1, br1, Wt1, Wr2, br2, Wt2, Wr3, br3, Wt3,
           Wr4, br4, Wt4, Wr5, br5, Wt5, Wr6, br6, Wt6, Wr7, br7, Wt7):
    B = x.shape[0]
    BB = 64
    while B % BB:
        BB //= 2
    grid = (B // BB,)

    weights = [Wg, bg.reshape(1, -1), Wb, bb.reshape(1, -1)]
    for Wr, br, Wt in ((Wr1, br1, Wt1), (Wr2, br2, Wt2), (Wr3, br3, Wt3),
                       (Wr4, br4, Wt4), (Wr5, br5, Wt5), (Wr6, br6, Wt6),
                       (Wr7, br7, Wt7)):
        weights += [Wr, br.reshape(1, -1), Wt]

    in_specs = [pl.BlockSpec((BB, _CH, x.shape[2]), lambda i: (i, 0, 0))]
    for w in weights:
        in_specs.append(pl.BlockSpec(w.shape, lambda i: (0,) * w.ndim))

    out_f = Wr7.shape[1]
    fn = lambda *refs: _body(refs, BB)
    return pl.pallas_call(
        fn,
        grid=grid,
        in_specs=in_specs,
        out_specs=pl.BlockSpec((BB, _CH, out_f), lambda i: (i, 0, 0)),
        out_shape=jax.ShapeDtypeStruct((B, _CH, out_f), jnp.float32),
    )(x, *weights)


# speedup vs baseline: 15.6395x; 1.0006x over previous
import jax
import jax.numpy as jnp
from jax import lax
from jax.experimental import pallas as pl

_CH = 62
_CHP = 64
_TOPK = 10


def _mm(a, b):
    return jnp.dot(a, b, preferred_element_type=jnp.float32)


def _bmm(a, b):
    return lax.dot_general(a, b, (((2,), (1,)), ((0,), (0,))),
                           preferred_element_type=jnp.float32)


def _body(refs, bb):
    (x_ref, Wg, bg, Wb, bbias,
     Wr1, br1, Wt1, Wr2, br2, Wt2, Wr3, br3, Wt3, Wr4, br4, Wt4,
     Wr5, br5, Wt5, Wr6, br6, Wt6, Wr7, br7, Wt7, o_ref) = refs

    xv = x_ref[...]
    xp = jnp.concatenate(
        [xv, jnp.zeros((bb, _CHP - _CH, xv.shape[2]), jnp.float32)], axis=1)
    x2 = xp.reshape(bb * _CHP, xv.shape[2])

    xm = jnp.tanh(_mm(x2, Wg[...]) + bg[...])
    xa = jnp.tanh(_mm(x2, Wb[...]) + bbias[...])
    xa3 = xa.reshape(bb, _CHP, xa.shape[1])

    adj = lax.dot_general(xa3, xa3, (((2,), (2,)), ((0,), (0,))),
                          preferred_element_type=jnp.float32)
    colidx = lax.broadcasted_iota(jnp.int32, (1, 1, _CHP), 2)
    e = jnp.where(colidx < _CH, jnp.exp(adj - 64.0), 0.0)
    work = e / jnp.sum(e, axis=2, keepdims=True)

    for _ in range(_TOPK):
        m = jnp.max(work, axis=2, keepdims=True)
        m = jnp.where(m >= 0.0, m, 100.0)
        work = jnp.where(work == m, work - 2.0, work)
    adjm = jnp.where(work < 0.0, work + 2.0, 0.0)

    def gconv(h2, Wr, br, Wt):
        f = h2.shape[1]
        h3 = h2.reshape(bb, _CHP, f)
        a2 = _bmm(adjm, h3).reshape(bb * _CHP, f)
        return _mm(a2, Wr[...]) + br[...] + _mm(h2, Wt[...])

    h = jax.nn.relu(gconv(x2, Wr1, br1, Wt1))
    h = h + jax.nn.relu(gconv(h, Wr2, br2, Wt2))
    h = h + jax.nn.relu(gconv(h, Wr3, br3, Wt3))
    h = h + jax.nn.relu(gconv(h, Wr4, br4, Wt4))
    h = h + jax.nn.relu(gconv(h, Wr5, br5, Wt5))
    h = h + jax.nn.relu(gconv(h, Wr6, br6, Wt6))
    h = jax.nn.relu(gconv(h, Wr7, br7, Wt7))

    out = (h * xm).reshape(bb, _CHP, xm.shape[1])
    o_ref[...] = out[:, :_CH, :]


def kernel(x, Wg, bg, Wb, bb, Wr1, br1, Wt1, Wr2, br2, Wt2, Wr3, br3, Wt3,
           Wr4, br4, Wt4, Wr5, br5, Wt5, Wr6, br6, Wt6, Wr7, br7, Wt7):
    B = x.shape[0]
    BB = 64
    while B % BB:
        BB //= 2
    grid = (B // BB,)

    weights = [Wg, bg.reshape(1, -1), Wb, bb.reshape(1, -1)]
    for Wr, br, Wt in ((Wr1, br1, Wt1), (Wr2, br2, Wt2), (Wr3, br3, Wt3),
                       (Wr4, br4, Wt4), (Wr5, br5, Wt5), (Wr6, br6, Wt6),
                       (Wr7, br7, Wt7)):
        weights += [Wr, br.reshape(1, -1), Wt]

    in_specs = [pl.BlockSpec((BB, _CH, x.shape[2]), lambda i: (i, 0, 0))]
    for w in weights:
        in_specs.append(pl.BlockSpec(w.shape, lambda i: (0,) * w.ndim))

    out_f = Wr7.shape[1]
    fn = lambda *refs: _body(refs, BB)
    return pl.pallas_call(
        fn,
        grid=grid,
        in_specs=in_specs,
        out_specs=pl.BlockSpec((BB, _CH, out_f), lambda i: (i, 0, 0)),
        out_shape=jax.ShapeDtypeStruct((B, _CH, out_f), jnp.float32),
    )(x, *weights)


# speedup vs baseline: 18.5186x; 1.1841x over previous
import jax
import jax.numpy as jnp
from jax import lax
from jax.experimental import pallas as pl

_CH = 62
_CHP = 64
_TOPK = 10


def _mm(a, b):
    return jnp.dot(a, b, preferred_element_type=jnp.float32)


def _bmm(a, b):
    return lax.dot_general(a, b, (((2,), (1,)), ((0,), (0,))),
                           preferred_element_type=jnp.float32)


def _body(refs, bb):
    (x_ref, Wg, bg, Wb, bbias,
     Wr1, br1, Wt1, Wr2, br2, Wt2, Wr3, br3, Wt3, Wr4, br4, Wt4,
     Wr5, br5, Wt5, Wr6, br6, Wt6, Wr7, br7, Wt7, o_ref) = refs

    xv = x_ref[...]
    xp = jnp.concatenate(
        [xv, jnp.zeros((bb, _CHP - _CH, xv.shape[2]), jnp.float32)], axis=1)
    x2 = xp.reshape(bb * _CHP, xv.shape[2])

    xm = jnp.tanh(_mm(x2, Wg[...]) + bg[...])
    xa = jnp.tanh(_mm(x2, Wb[...]) + bbias[...])

    xap = xa.reshape(bb // 2, 2 * _CHP, xa.shape[1])
    bd = lax.dot_general(xap, xap, (((2,), (2,)), ((0,), (0,))),
                         preferred_element_type=jnp.float32)

    lane = lax.broadcasted_iota(jnp.int32, (1, 1, 2 * _CHP), 2)
    E = jnp.where(lane < _CHP, bd[:, :_CHP, :], bd[:, _CHP:, :])
    jrow = lax.broadcasted_iota(jnp.int32, (1, _CHP, 1), 1)
    e = jnp.where(jrow < _CH, jnp.exp(E - 64.0), 0.0)
    S = jnp.sum(e, axis=1, keepdims=True)

    work = e
    for _ in range(_TOPK):
        m = jnp.max(work, axis=1, keepdims=True)
        m = jnp.where(m >= 0.0, m, 100.0)
        work = jnp.where(work == m, work - 2.0, work)
    Gp = jnp.where(work < 0.0, e, 0.0) / S

    Gbd = jnp.concatenate([jnp.where(lane < _CHP, Gp, 0.0),
                           jnp.where(lane >= _CHP, Gp, 0.0)], axis=1)

    def gconv(h2, Wr, br, Wt):
        f = h2.shape[1]
        hp = h2.reshape(bb // 2, 2 * _CHP, f)
        a3 = lax.dot_general(Gbd, hp, (((1,), (1,)), ((0,), (0,))),
                             preferred_element_type=jnp.float32)
        a2 = a3.reshape(bb * _CHP, f)
        return _mm(a2, Wr[...]) + br[...] + _mm(h2, Wt[...])

    h = jax.nn.relu(gconv(x2, Wr1, br1, Wt1))
    h = h + jax.nn.relu(gconv(h, Wr2, br2, Wt2))
    h = h + jax.nn.relu(gconv(h, Wr3, br3, Wt3))
    h = h + jax.nn.relu(gconv(h, Wr4, br4, Wt4))
    h = h + jax.nn.relu(gconv(h, Wr5, br5, Wt5))
    h = h + jax.nn.relu(gconv(h, Wr6, br6, Wt6))
    h = jax.nn.relu(gconv(h, Wr7, br7, Wt7))

    out = (h * xm).reshape(bb, _CHP, xm.shape[1])
    o_ref[...] = out[:, :_CH, :]


def kernel(x, Wg, bg, Wb, bb, Wr1, br1, Wt1, Wr2, br2, Wt2, Wr3, br3, Wt3,
           Wr4, br4, Wt4, Wr5, br5, Wt5, Wr6, br6, Wt6, Wr7, br7, Wt7):
    B = x.shape[0]
    BB = 64
    while B % BB:
        BB //= 2
    grid = (B // BB,)

    weights = [Wg, bg.reshape(1, -1), Wb, bb.reshape(1, -1)]
    for Wr, br, Wt in ((Wr1, br1, Wt1), (Wr2, br2, Wt2), (Wr3, br3, Wt3),
                       (Wr4, br4, Wt4), (Wr5, br5, Wt5), (Wr6, br6, Wt6),
                       (Wr7, br7, Wt7)):
        weights += [Wr, br.reshape(1, -1), Wt]

    in_specs = [pl.BlockSpec((BB, _CH, x.shape[2]), lambda i: (i, 0, 0))]
    for w in weights:
        in_specs.append(pl.BlockSpec(w.shape, lambda i: (0,) * w.ndim))

    out_f = Wr7.shape[1]
    fn = lambda *refs: _body(refs, BB)
    return pl.pallas_call(
        fn,
        grid=grid,
        in_specs=in_specs,
        out_specs=pl.BlockSpec((BB, _CH, out_f), lambda i: (i, 0, 0)),
        out_shape=jax.ShapeDtypeStruct((B, _CH, out_f), jnp.float32),
    )(x, *weights)


# speedup vs baseline: 19.2335x; 1.0386x over previous
import jax
import jax.numpy as jnp
from jax import lax
from jax.experimental import pallas as pl

_CH = 62
_CHP = 64
_TOPK = 10


def _mm(a, b):
    return jnp.dot(a, b, preferred_element_type=jnp.float32)


def _bmm(a, b):
    return lax.dot_general(a, b, (((2,), (1,)), ((0,), (0,))),
                           preferred_element_type=jnp.float32)


def _body(refs, bb):
    (x_ref, Wg, bg, Wb, bbias,
     Wr1, br1, Wt1, Wr2, br2, Wt2, Wr3, br3, Wt3, Wr4, br4, Wt4,
     Wr5, br5, Wt5, Wr6, br6, Wt6, Wr7, br7, Wt7, o_ref) = refs

    xv = x_ref[...]
    xp = jnp.concatenate(
        [xv, jnp.zeros((bb, _CHP - _CH, xv.shape[2]), jnp.float32)], axis=1)
    x2 = xp.reshape(bb * _CHP, xv.shape[2])

    xm = jnp.tanh(_mm(x2, Wg[...]) + bg[...])
    xa = jnp.tanh(_mm(x2, Wb[...]) + bbias[...])

    xap = xa.reshape(bb // 2, 2 * _CHP, xa.shape[1])
    bd = lax.dot_general(xap, xap, (((2,), (2,)), ((0,), (0,))),
                         preferred_element_type=jnp.float32)

    lane = lax.broadcasted_iota(jnp.int32, (1, 1, 2 * _CHP), 2)
    E = jnp.where(lane < _CHP, bd[:, :_CHP, :], bd[:, _CHP:, :])
    jrow = lax.broadcasted_iota(jnp.int32, (1, _CHP, 1), 1)
    e = jnp.where(jrow < _CH, jnp.exp(E - 64.0), 0.0)
    S = jnp.sum(e, axis=1, keepdims=True)

    work = e
    for _ in range(_TOPK):
        m = jnp.max(work, axis=1, keepdims=True)
        m = jnp.where(m >= 0.0, m, 100.0)
        work = jnp.where(work == m, work - 2.0, work)
    Gp = jnp.where(work < 0.0, e, 0.0) / S

    Gbd = jnp.concatenate([jnp.where(lane < _CHP, Gp, 0.0),
                           jnp.where(lane >= _CHP, Gp, 0.0)], axis=1)

    def gconv(h2, Wr, br, Wt):
        f = h2.shape[1]
        hp = h2.reshape(bb // 2, 2 * _CHP, f)
        a3 = lax.dot_general(Gbd, hp, (((1,), (1,)), ((0,), (0,))),
                             preferred_element_type=jnp.float32)
        a2 = a3.reshape(bb * _CHP, f)
        return _mm(a2, Wr[...]) + br[...] + _mm(h2, Wt[...])

    h = jax.nn.relu(gconv(x2, Wr1, br1, Wt1))
    h = h + jax.nn.relu(gconv(h, Wr2, br2, Wt2))
    h = h + jax.nn.relu(gconv(h, Wr3, br3, Wt3))
    h = h + jax.nn.relu(gconv(h, Wr4, br4, Wt4))
    h = h + jax.nn.relu(gconv(h, Wr5, br5, Wt5))
    h = h + jax.nn.relu(gconv(h, Wr6, br6, Wt6))
    h = jax.nn.relu(gconv(h, Wr7, br7, Wt7))

    out = (h * xm).reshape(bb, _CHP, xm.shape[1])
    o_ref[...] = out[:, :_CH, :]


def kernel(x, Wg, bg, Wb, bb, Wr1, br1, Wt1, Wr2, br2, Wt2, Wr3, br3, Wt3,
           Wr4, br4, Wt4, Wr5, br5, Wt5, Wr6, br6, Wt6, Wr7, br7, Wt7):
    B = x.shape[0]
    BB = 128
    while B % BB:
        BB //= 2
    grid = (B // BB,)

    weights = [Wg, bg.reshape(1, -1), Wb, bb.reshape(1, -1)]
    for Wr, br, Wt in ((Wr1, br1, Wt1), (Wr2, br2, Wt2), (Wr3, br3, Wt3),
                       (Wr4, br4, Wt4), (Wr5, br5, Wt5), (Wr6, br6, Wt6),
                       (Wr7, br7, Wt7)):
        weights += [Wr, br.reshape(1, -1), Wt]

    in_specs = [pl.BlockSpec((BB, _CH, x.shape[2]), lambda i: (i, 0, 0))]
    for w in weights:
        in_specs.append(pl.BlockSpec(w.shape, lambda i: (0,) * w.ndim))

    out_f = Wr7.shape[1]
    fn = lambda *refs: _body(refs, BB)
    return pl.pallas_call(
        fn,
        grid=grid,
        in_specs=in_specs,
        out_specs=pl.BlockSpec((BB, _CH, out_f), lambda i: (i, 0, 0)),
        out_shape=jax.ShapeDtypeStruct((B, _CH, out_f), jnp.float32),
    )(x, *weights)
